# baseline (device time: 88126 ns/iter reference)
import jax
import jax.numpy as jnp
from jax import lax
from jax.experimental import pallas as pl
from jax.experimental.pallas import tpu as pltpu

N_GLOBAL = 4096
EPS = 1e-5
CHUNK = 512
NBUF = 3


def kernel(x, gamma, beta):
    m, n = x.shape
    nchunks = m // CHUNK

    def body(x_hbm, g_ref, b_ref, o_ref, stage, stats, rstats,
             in_sems, send_sem, recv_sem):
        my_x = lax.axis_index("x")
        my_y = lax.axis_index("y")
        peer = (my_x, 1 - my_y)

        barrier = pltpu.get_barrier_semaphore()
        pl.semaphore_signal(barrier, inc=1, device_id=peer,
                            device_id_type=pl.DeviceIdType.MESH)
        pl.semaphore_wait(barrier, 1)

        def make_load(i, slot):
            return pltpu.make_async_copy(
                x_hbm.at[pl.ds(i * CHUNK, CHUNK), :],
                stage.at[slot],
                in_sems.at[slot],
            )

        ones = jnp.ones((n, 128), jnp.bfloat16)

        loads = {}
        for j in range(min(NBUF, nchunks)):
            loads[j] = make_load(j, j)
            loads[j].start()
        for i in range(nchunks):
            slot = i % NBUF
            loads[slot].wait()
            rows = pl.ds(i * CHUNK, CHUNK)
            cb = stage[slot].astype(jnp.bfloat16)
            o_ref[rows, :] = cb
            s1 = lax.dot_general(
                cb, ones, (((1,), (0,)), ((), ())),
                preferred_element_type=jnp.float32)
            s2 = lax.dot_general(
                cb * cb, ones, (((1,), (0,)), ((), ())),
                preferred_element_type=jnp.float32)
            stats[rows, 0:1] = s1[:, 0:1]
            stats[rows, 1:2] = s2[:, 0:1]
            if i + NBUF < nchunks:
                loads[slot] = make_load(i + NBUF, slot)
                loads[slot].start()

        rdma = pltpu.make_async_remote_copy(
            src_ref=stats,
            dst_ref=rstats,
            send_sem=send_sem,
            recv_sem=recv_sem,
            device_id=peer,
            device_id_type=pl.DeviceIdType.MESH,
        )
        rdma.start()
        rdma.wait()

        gb = g_ref[...].astype(jnp.bfloat16)
        bb = b_ref[...].astype(jnp.bfloat16)

        for i in range(nchunks):
            rows = pl.ds(i * CHUNK, CHUNK)
            st = stats[rows, :] + rstats[rows, :]
            mean = st[:, 0:1] * (1.0 / N_GLOBAL)
            var = st[:, 1:2] * (1.0 / N_GLOBAL) - mean * mean
            inv = lax.rsqrt(var + EPS)
            meanb = mean.astype(jnp.bfloat16)
            invb = inv.astype(jnp.bfloat16)
            cb = o_ref[rows, :]
            o_ref[rows, :] = ((cb - meanb) * invb) * gb + bb

    return pl.pallas_call(
        body,
        out_shape=jax.ShapeDtypeStruct((m, n), jnp.bfloat16),
        in_specs=[
            pl.BlockSpec(memory_space=pl.ANY),
            pl.BlockSpec(memory_space=pltpu.VMEM),
            pl.BlockSpec(memory_space=pltpu.VMEM),
        ],
        out_specs=pl.BlockSpec(memory_space=pltpu.VMEM),
        scratch_shapes=[
            pltpu.VMEM((NBUF, CHUNK, n), jnp.float32),
            pltpu.VMEM((m, 2), jnp.float32),
            pltpu.VMEM((m, 2), jnp.float32),
            pltpu.SemaphoreType.DMA((NBUF,)),
            pltpu.SemaphoreType.DMA,
            pltpu.SemaphoreType.DMA,
        ],
        compiler_params=pltpu.CompilerParams(
            collective_id=0,
            vmem_limit_bytes=60 * 1024 * 1024,
        ),
    )(x, gamma.reshape(1, n), beta.reshape(1, n))


# device time: 57151 ns/iter; 1.5420x vs baseline; 1.5420x over previous
import jax
import jax.numpy as jnp
from jax import lax
from jax.experimental import pallas as pl
from jax.experimental.pallas import tpu as pltpu

N_GLOBAL = 4096
EPS = 1e-5
CHUNK = 512
NBUF = 3
BLK = CHUNK // 128


def kernel(x, gamma, beta):
    m, n = x.shape
    nchunks = m // CHUNK
    srows = m // 128

    def body(x_hbm, g_ref, b_ref, o_ref, stage, stats, rstats, sbuf, rsbuf,
             in_sems, send_sem, recv_sem):
        my_x = lax.axis_index("x")
        my_y = lax.axis_index("y")
        peer = (my_x, 1 - my_y)

        barrier = pltpu.get_barrier_semaphore()
        pl.semaphore_signal(barrier, inc=1, device_id=peer,
                            device_id_type=pl.DeviceIdType.MESH)
        pl.semaphore_wait(barrier, 1)

        def make_load(i, slot):
            return pltpu.make_async_copy(
                x_hbm.at[pl.ds(i * CHUNK, CHUNK), :],
                stage.at[slot],
                in_sems.at[slot],
            )

        ones = jnp.ones((n, 128), jnp.bfloat16)
        row = lax.broadcasted_iota(jnp.int32, (CHUNK, 128), 0)
        col = lax.broadcasted_iota(jnp.int32, (CHUNK, 128), 1)
        eyecol = (col == row % 128).astype(jnp.float32)
        brow = lax.broadcasted_iota(jnp.int32, (BLK, CHUNK), 0)
        bcol = lax.broadcasted_iota(jnp.int32, (BLK, CHUNK), 1)
        blockmask = (bcol // 128 == brow).astype(jnp.float32)
        expandmask = blockmask.T
        ones_col = jnp.ones((128, 1), jnp.float32)

        def pack(s, i, base):
            sbuf[pl.ds(base + i * BLK, BLK), :] = lax.dot_general(
                blockmask, s * eyecol, (((1,), (0,)), ((), ())),
                preferred_element_type=jnp.float32)

        loads = {}
        for j in range(min(NBUF, nchunks)):
            loads[j] = make_load(j, j)
            loads[j].start()
        for i in range(nchunks):
            slot = i % NBUF
            loads[slot].wait()
            rows = pl.ds(i * CHUNK, CHUNK)
            cb = stage[slot].astype(jnp.bfloat16)
            o_ref[rows, :] = cb
            s1 = lax.dot_general(
                cb, ones, (((1,), (0,)), ((), ())),
                preferred_element_type=jnp.float32)
            s2 = lax.dot_general(
                cb * cb, ones, (((1,), (0,)), ((), ())),
                preferred_element_type=jnp.float32)
            stats[rows, 0:1] = s1[:, 0:1]
            stats[rows, 1:2] = s2[:, 0:1]
            pack(s1, i, 0)
            pack(s2, i, srows)
            if i + NBUF < nchunks:
                loads[slot] = make_load(i + NBUF, slot)
                loads[slot].start()

        rdma = pltpu.make_async_remote_copy(
            src_ref=sbuf,
            dst_ref=rsbuf,
            send_sem=send_sem,
            recv_sem=recv_sem,
            device_id=peer,
            device_id_type=pl.DeviceIdType.MESH,
        )
        rdma.start()
        rdma.wait()

        for i in range(nchunks):
            rows = pl.ds(i * CHUNK, CHUNK)
            for base, colidx in ((0, 0), (srows, 1)):
                blk = rsbuf[pl.ds(base + i * BLK, BLK), :]
                expanded = lax.dot_general(
                    expandmask, blk, (((1,), (0,)), ((), ())),
                    preferred_element_type=jnp.float32)
                rstats[rows, colidx:colidx + 1] = lax.dot_general(
                    expanded * eyecol, ones_col, (((1,), (0,)), ((), ())),
                    preferred_element_type=jnp.float32)

        gb = g_ref[...].astype(jnp.bfloat16)
        bb = b_ref[...].astype(jnp.bfloat16)

        for i in range(nchunks):
            rows = pl.ds(i * CHUNK, CHUNK)
            st = stats[rows, :] + rstats[rows, :]
            mean = st[:, 0:1] * (1.0 / N_GLOBAL)
            var = st[:, 1:2] * (1.0 / N_GLOBAL) - mean * mean
            inv = lax.rsqrt(var + EPS)
            meanb = mean.astype(jnp.bfloat16)
            invb = inv.astype(jnp.bfloat16)
            cb = o_ref[rows, :]
            o_ref[rows, :] = ((cb - meanb) * invb) * gb + bb

    return pl.pallas_call(
        body,
        out_shape=jax.ShapeDtypeStruct((m, n), jnp.bfloat16),
        in_specs=[
            pl.BlockSpec(memory_space=pl.ANY),
            pl.BlockSpec(memory_space=pltpu.VMEM),
            pl.BlockSpec(memory_space=pltpu.VMEM),
        ],
        out_specs=pl.BlockSpec(memory_space=pltpu.VMEM),
        scratch_shapes=[
            pltpu.VMEM((NBUF, CHUNK, n), jnp.float32),
            pltpu.VMEM((m, 2), jnp.float32),
            pltpu.VMEM((m, 2), jnp.float32),
            pltpu.VMEM((2 * (m // 128), 128), jnp.float32),
            pltpu.VMEM((2 * (m // 128), 128), jnp.float32),
            pltpu.SemaphoreType.DMA((NBUF,)),
            pltpu.SemaphoreType.DMA,
            pltpu.SemaphoreType.DMA,
        ],
        compiler_params=pltpu.CompilerParams(
            collective_id=0,
            vmem_limit_bytes=60 * 1024 * 1024,
        ),
    )(x, gamma.reshape(1, n), beta.reshape(1, n))


# device time: 56191 ns/iter; 1.5683x vs baseline; 1.0171x over previous
import jax
import jax.numpy as jnp
from jax import lax
from jax.experimental import pallas as pl
from jax.experimental.pallas import tpu as pltpu

N_GLOBAL = 4096
EPS = 1e-5
CHUNK = 512
NBUF = 4
BLK = CHUNK // 128


def kernel(x, gamma, beta):
    m, n = x.shape
    nchunks = m // CHUNK
    srows = m // 128

    def body(x_hbm, g_ref, b_ref, o_ref, stage, sbuf, rsbuf,
             in_sems, send_sem, recv_sem):
        my_x = lax.axis_index("x")
        my_y = lax.axis_index("y")
        peer = (my_x, 1 - my_y)

        barrier = pltpu.get_barrier_semaphore()
        pl.semaphore_signal(barrier, inc=1, device_id=peer,
                            device_id_type=pl.DeviceIdType.MESH)
        pl.semaphore_wait(barrier, 1)

        def make_load(i, slot):
            return pltpu.make_async_copy(
                x_hbm.at[pl.ds(i * CHUNK, CHUNK), :],
                stage.at[slot],
                in_sems.at[slot],
            )

        ones = jnp.ones((n, 128), jnp.bfloat16)
        row = lax.broadcasted_iota(jnp.int32, (CHUNK, 128), 0)
        col = lax.broadcasted_iota(jnp.int32, (CHUNK, 128), 1)
        eyecol = (col == row % 128).astype(jnp.float32)
        brow = lax.broadcasted_iota(jnp.int32, (BLK, CHUNK), 0)
        bcol = lax.broadcasted_iota(jnp.int32, (BLK, CHUNK), 1)
        blockmask = (bcol // 128 == brow).astype(jnp.float32)
        expandmask = blockmask.T
        ones_col = jnp.ones((128, 1), jnp.float32)

        def pack(s, i, base):
            sbuf[pl.ds(base + i * BLK, BLK), :] = lax.dot_general(
                blockmask, s * eyecol, (((1,), (0,)), ((), ())),
                preferred_element_type=jnp.float32)

        def expand_col(packed, i):
            blk = packed[i * BLK:(i + 1) * BLK, :]
            expanded = lax.dot_general(
                expandmask, blk, (((1,), (0,)), ((), ())),
                preferred_element_type=jnp.float32)
            return lax.dot_general(
                expanded * eyecol, ones_col, (((1,), (0,)), ((), ())),
                preferred_element_type=jnp.float32)

        loads = {}
        for j in range(min(NBUF, nchunks)):
            loads[j] = make_load(j, j)
            loads[j].start()
        for i in range(nchunks):
            slot = i % NBUF
            loads[slot].wait()
            rows = pl.ds(i * CHUNK, CHUNK)
            cb = stage[slot].astype(jnp.bfloat16)
            o_ref[rows, :] = cb
            s1 = lax.dot_general(
                cb, ones, (((1,), (0,)), ((), ())),
                preferred_element_type=jnp.float32)
            s2 = lax.dot_general(
                cb * cb, ones, (((1,), (0,)), ((), ())),
                preferred_element_type=jnp.float32)
            pack(s1, i, 0)
            pack(s2, i, srows)
            if i + NBUF < nchunks:
                loads[slot] = make_load(i + NBUF, slot)
                loads[slot].start()

        rdma = pltpu.make_async_remote_copy(
            src_ref=sbuf,
            dst_ref=rsbuf,
            send_sem=send_sem,
            recv_sem=recv_sem,
            device_id=peer,
            device_id_type=pl.DeviceIdType.MESH,
        )
        rdma.start()
        rdma.wait()

        sums = sbuf[0:srows, :] + rsbuf[0:srows, :]
        sqs = sbuf[srows:2 * srows, :] + rsbuf[srows:2 * srows, :]
        meanp = sums * (1.0 / N_GLOBAL)
        varp = sqs * (1.0 / N_GLOBAL) - meanp * meanp
        invp = lax.rsqrt(varp + EPS)

        gb = g_ref[...].astype(jnp.bfloat16)
        bb = b_ref[...].astype(jnp.bfloat16)

        for i in range(nchunks):
            rows = pl.ds(i * CHUNK, CHUNK)
            meanb = expand_col(meanp, i).astype(jnp.bfloat16)
            invb = expand_col(invp, i).astype(jnp.bfloat16)
            cb = o_ref[rows, :]
            o_ref[rows, :] = ((cb - meanb) * invb) * gb + bb

    return pl.pallas_call(
        body,
        out_shape=jax.ShapeDtypeStruct((m, n), jnp.bfloat16),
        in_specs=[
            pl.BlockSpec(memory_space=pl.ANY),
            pl.BlockSpec(memory_space=pltpu.VMEM),
            pl.BlockSpec(memory_space=pltpu.VMEM),
        ],
        out_specs=pl.BlockSpec(memory_space=pltpu.VMEM),
        scratch_shapes=[
            pltpu.VMEM((NBUF, CHUNK, n), jnp.float32),
            pltpu.VMEM((2 * (m // 128), 128), jnp.float32),
            pltpu.VMEM((2 * (m // 128), 128), jnp.float32),
            pltpu.SemaphoreType.DMA((NBUF,)),
            pltpu.SemaphoreType.DMA,
            pltpu.SemaphoreType.DMA,
        ],
        compiler_params=pltpu.CompilerParams(
            collective_id=0,
            vmem_limit_bytes=60 * 1024 * 1024,
        ),
    )(x, gamma.reshape(1, n), beta.reshape(1, n))


# device time: 54445 ns/iter; 1.6186x vs baseline; 1.0321x over previous
import jax
import jax.numpy as jnp
from jax import lax
from jax.experimental import pallas as pl
from jax.experimental.pallas import tpu as pltpu

N_GLOBAL = 4096
EPS = 1e-5
CHUNK = 512
NBUF = 4
BLK = CHUNK // 128
PB = 2 * BLK


def kernel(x, gamma, beta):
    m, n = x.shape
    nchunks = m // CHUNK
    half = nchunks // 2

    def body(x_hbm, g_ref, b_ref, o_hbm, stage, xb, sbuf, rsbuf,
             in_sems, out_sems, send_sems, recv_sems):
        my_x = lax.axis_index("x")
        my_y = lax.axis_index("y")
        peer = (my_x, 1 - my_y)

        barrier = pltpu.get_barrier_semaphore()
        pl.semaphore_signal(barrier, inc=1, device_id=peer,
                            device_id_type=pl.DeviceIdType.MESH)
        pl.semaphore_wait(barrier, 1)

        def make_load(i, slot):
            return pltpu.make_async_copy(
                x_hbm.at[pl.ds(i * CHUNK, CHUNK), :],
                stage.at[slot],
                in_sems.at[slot],
            )

        def half_rdma(h):
            rows = pl.ds(h * half * PB, half * PB)
            return pltpu.make_async_remote_copy(
                src_ref=sbuf.at[rows, :],
                dst_ref=rsbuf.at[rows, :],
                send_sem=send_sems.at[h],
                recv_sem=recv_sems.at[h],
                device_id=peer,
                device_id_type=pl.DeviceIdType.MESH,
            )

        ones = jnp.ones((n, 128), jnp.bfloat16)
        row = lax.broadcasted_iota(jnp.int32, (CHUNK, 128), 0)
        col = lax.broadcasted_iota(jnp.int32, (CHUNK, 128), 1)
        eyecol = (col == row % 128).astype(jnp.float32)
        brow = lax.broadcasted_iota(jnp.int32, (BLK, CHUNK), 0)
        bcol = lax.broadcasted_iota(jnp.int32, (BLK, CHUNK), 1)
        blockmask = (bcol // 128 == brow).astype(jnp.float32)
        expandmask = blockmask.T
        ones_col = jnp.ones((128, 1), jnp.float32)

        def pack(s, base):
            sbuf[pl.ds(base, BLK), :] = lax.dot_general(
                blockmask, s * eyecol, (((1,), (0,)), ((), ())),
                preferred_element_type=jnp.float32)

        def expand_col(blk):
            expanded = lax.dot_general(
                expandmask, blk, (((1,), (0,)), ((), ())),
                preferred_element_type=jnp.float32)
            return lax.dot_general(
                expanded * eyecol, ones_col, (((1,), (0,)), ((), ())),
                preferred_element_type=jnp.float32)

        rdmas = {}
        loads = {}
        for j in range(min(NBUF, nchunks)):
            loads[j] = make_load(j, j)
            loads[j].start()
        for i in range(nchunks):
            slot = i % NBUF
            loads[slot].wait()
            rows = pl.ds(i * CHUNK, CHUNK)
            cb = stage[slot].astype(jnp.bfloat16)
            xb[rows, :] = cb
            s1 = lax.dot_general(
                cb, ones, (((1,), (0,)), ((), ())),
                preferred_element_type=jnp.float32)
            s2 = lax.dot_general(
                cb * cb, ones, (((1,), (0,)), ((), ())),
                preferred_element_type=jnp.float32)
            pack(s1, i * PB)
            pack(s2, i * PB + BLK)
            if i == half - 1:
                rdmas[0] = half_rdma(0)
                rdmas[0].start()
            if i == nchunks - 1:
                rdmas[1] = half_rdma(1)
                rdmas[1].start()
            if i + NBUF < nchunks:
                loads[slot] = make_load(i + NBUF, slot)
                loads[slot].start()
        rdmas[0].wait()
        rdmas[1].wait()

        gb = g_ref[...].astype(jnp.bfloat16)
        bb = b_ref[...].astype(jnp.bfloat16)

        stores = {}
        for i in range(nchunks):
            if i >= 2:
                stores[i % 2].wait()
            rows = pl.ds(i * CHUNK, CHUNK)
            sblk = sbuf[pl.ds(i * PB, BLK), :] + rsbuf[pl.ds(i * PB, BLK), :]
            qblk = (sbuf[pl.ds(i * PB + BLK, BLK), :]
                    + rsbuf[pl.ds(i * PB + BLK, BLK), :])
            meanp = sblk * (1.0 / N_GLOBAL)
            varp = qblk * (1.0 / N_GLOBAL) - meanp * meanp
            invp = lax.rsqrt(varp + EPS)
            meanb = expand_col(meanp).astype(jnp.bfloat16)
            invb = expand_col(invp).astype(jnp.bfloat16)
            cb = xb[rows, :]
            xb[rows, :] = ((cb - meanb) * invb) * gb + bb
            stores[i % 2] = pltpu.make_async_copy(
                xb.at[rows, :],
                o_hbm.at[rows, :],
                out_sems.at[i % 2],
            )
            stores[i % 2].start()
        stores[0].wait()
        stores[1].wait()

    return pl.pallas_call(
        body,
        out_shape=jax.ShapeDtypeStruct((m, n), jnp.bfloat16),
        in_specs=[
            pl.BlockSpec(memory_space=pl.ANY),
            pl.BlockSpec(memory_space=pltpu.VMEM),
            pl.BlockSpec(memory_space=pltpu.VMEM),
        ],
        out_specs=pl.BlockSpec(memory_space=pl.ANY),
        scratch_shapes=[
            pltpu.VMEM((NBUF, CHUNK, n), jnp.float32),
            pltpu.VMEM((m, n), jnp.bfloat16),
            pltpu.VMEM((m // 128 * 2, 128), jnp.float32),
            pltpu.VMEM((m // 128 * 2, 128), jnp.float32),
            pltpu.SemaphoreType.DMA((NBUF,)),
            pltpu.SemaphoreType.DMA((2,)),
            pltpu.SemaphoreType.DMA((2,)),
            pltpu.SemaphoreType.DMA((2,)),
        ],
        compiler_params=pltpu.CompilerParams(
            collective_id=0,
            vmem_limit_bytes=60 * 1024 * 1024,
        ),
    )(x, gamma.reshape(1, n), beta.reshape(1, n))


# device time: 50866 ns/iter; 1.7325x vs baseline; 1.0704x over previous
import jax
import jax.numpy as jnp
from jax import lax
from jax.experimental import pallas as pl
from jax.experimental.pallas import tpu as pltpu

N_GLOBAL = 4096
EPS = 1e-5
CHUNK = 512
NBUF = 4
DELAY = 3
BLK = CHUNK // 128
PB = 2 * BLK


def kernel(x, gamma, beta):
    m, n = x.shape
    nchunks = m // CHUNK

    def body(x_hbm, g_ref, b_ref, o_hbm, stage, xb, sbuf, rsbuf,
             in_sems, out_sems, send_sems, recv_sems):
        my_x = lax.axis_index("x")
        my_y = lax.axis_index("y")
        peer = (my_x, 1 - my_y)

        barrier = pltpu.get_barrier_semaphore()
        pl.semaphore_signal(barrier, inc=1, device_id=peer,
                            device_id_type=pl.DeviceIdType.MESH)
        pl.semaphore_wait(barrier, 1)

        def make_load(i, slot):
            return pltpu.make_async_copy(
                x_hbm.at[pl.ds(i * CHUNK, CHUNK), :],
                stage.at[slot],
                in_sems.at[slot],
            )

        def chunk_rdma(i):
            rows = pl.ds(i * PB, PB)
            return pltpu.make_async_remote_copy(
                src_ref=sbuf.at[rows, :],
                dst_ref=rsbuf.at[rows, :],
                send_sem=send_sems.at[i],
                recv_sem=recv_sems.at[i],
                device_id=peer,
                device_id_type=pl.DeviceIdType.MESH,
            )

        ones = jnp.ones((n, 128), jnp.bfloat16)
        row = lax.broadcasted_iota(jnp.int32, (CHUNK, 128), 0)
        col = lax.broadcasted_iota(jnp.int32, (CHUNK, 128), 1)
        eyecol = (col == row % 128).astype(jnp.float32)
        brow = lax.broadcasted_iota(jnp.int32, (BLK, CHUNK), 0)
        bcol = lax.broadcasted_iota(jnp.int32, (BLK, CHUNK), 1)
        blockmask = (bcol // 128 == brow).astype(jnp.float32)
        expandmask = blockmask.T
        ones_col = jnp.ones((128, 1), jnp.float32)

        def pack(s, base):
            sbuf[pl.ds(base, BLK), :] = lax.dot_general(
                blockmask, s * eyecol, (((1,), (0,)), ((), ())),
                preferred_element_type=jnp.float32)

        def expand_col(blk):
            expanded = lax.dot_general(
                expandmask, blk, (((1,), (0,)), ((), ())),
                preferred_element_type=jnp.float32)
            return lax.dot_general(
                expanded * eyecol, ones_col, (((1,), (0,)), ((), ())),
                preferred_element_type=jnp.float32)

        gb = g_ref[...].astype(jnp.bfloat16)
        bb = b_ref[...].astype(jnp.bfloat16)

        rdmas = {}
        loads = {}
        stores = {}

        def produce(i):
            slot = i % NBUF
            loads[slot].wait()
            rows = pl.ds(i * CHUNK, CHUNK)
            cb = stage[slot].astype(jnp.bfloat16)
            xb[rows, :] = cb
            s1 = lax.dot_general(
                cb, ones, (((1,), (0,)), ((), ())),
                preferred_element_type=jnp.float32)
            s2 = lax.dot_general(
                cb * cb, ones, (((1,), (0,)), ((), ())),
                preferred_element_type=jnp.float32)
            pack(s1, i * PB)
            pack(s2, i * PB + BLK)
            rdmas[i] = chunk_rdma(i)
            rdmas[i].start()
            if i + NBUF < nchunks:
                loads[slot] = make_load(i + NBUF, slot)
                loads[slot].start()

        def consume(j):
            if j >= 2:
                stores[j % 2].wait()
            rdmas[j].wait()
            rows = pl.ds(j * CHUNK, CHUNK)
            sblk = sbuf[pl.ds(j * PB, BLK), :] + rsbuf[pl.ds(j * PB, BLK), :]
            qblk = (sbuf[pl.ds(j * PB + BLK, BLK), :]
                    + rsbuf[pl.ds(j * PB + BLK, BLK), :])
            meanp = sblk * (1.0 / N_GLOBAL)
            varp = qblk * (1.0 / N_GLOBAL) - meanp * meanp
            invp = lax.rsqrt(varp + EPS)
            meanb = expand_col(meanp).astype(jnp.bfloat16)
            invb = expand_col(invp).astype(jnp.bfloat16)
            cb = xb[rows, :]
            xb[rows, :] = ((cb - meanb) * invb) * gb + bb
            stores[j % 2] = pltpu.make_async_copy(
                xb.at[rows, :],
                o_hbm.at[rows, :],
                out_sems.at[j % 2],
            )
            stores[j % 2].start()

        for j in range(min(NBUF, nchunks)):
            loads[j] = make_load(j, j)
            loads[j].start()
        for i in range(nchunks + DELAY):
            if i < nchunks:
                produce(i)
            j = i - DELAY
            if 0 <= j < nchunks:
                consume(j)
        stores[0].wait()
        stores[1].wait()

    return pl.pallas_call(
        body,
        out_shape=jax.ShapeDtypeStruct((m, n), jnp.bfloat16),
        in_specs=[
            pl.BlockSpec(memory_space=pl.ANY),
            pl.BlockSpec(memory_space=pltpu.VMEM),
            pl.BlockSpec(memory_space=pltpu.VMEM),
        ],
        out_specs=pl.BlockSpec(memory_space=pl.ANY),
        scratch_shapes=[
            pltpu.VMEM((NBUF, CHUNK, n), jnp.float32),
            pltpu.VMEM((m, n), jnp.bfloat16),
            pltpu.VMEM((m // 128 * 2, 128), jnp.float32),
            pltpu.VMEM((m // 128 * 2, 128), jnp.float32),
            pltpu.SemaphoreType.DMA((NBUF,)),
            pltpu.SemaphoreType.DMA((2,)),
            pltpu.SemaphoreType.DMA((m // CHUNK,)),
            pltpu.SemaphoreType.DMA((m // CHUNK,)),
        ],
        compiler_params=pltpu.CompilerParams(
            collective_id=0,
            vmem_limit_bytes=60 * 1024 * 1024,
        ),
    )(x, gamma.reshape(1, n), beta.reshape(1, n))
